# baseline (speedup 1.0000x reference)
import functools

import jax
import jax.numpy as jnp
from jax import lax
from jax.experimental import pallas as pl
from jax.experimental.pallas import tpu as pltpu
from jax.experimental.pallas import tpu_sc as plsc

NY, NX, C, N, BS = 496, 432, 64, 40000, 4
NYNX = NY * NX
BASE = BS * NYNX
NTILES = 32
SEG = BASE // NTILES
SEG_V = SEG // 16
N_V = N // 16
NPAD = N + 16
SENT = N

_MESH = plsc.VectorSubcoreMesh(core_axis_name="c", subcore_axis_name="s")
_PARAMS = pltpu.CompilerParams(needs_layout_passes=False)


def _wid():
    return lax.axis_index("s") * 2 + lax.axis_index("c")


@functools.partial(
    pl.kernel,
    out_type=jax.ShapeDtypeStruct((BASE,), jnp.int32),
    mesh=_MESH,
    compiler_params=_PARAMS,
    scratch_types=[
        pltpu.VMEM((N,), jnp.int32),
        pltpu.VMEM((SEG,), jnp.int32),
    ],
)
def _build_inv(flat_hbm, inv_hbm, flat_v, inv_v):
    wid = _wid()
    lo = wid * SEG
    pltpu.sync_copy(flat_hbm, flat_v)

    sent = jnp.full((16,), SENT, jnp.int32)

    def fill(i, _):
        inv_v[pl.ds(i * 16, 16)] = sent
        return 0

    lax.fori_loop(0, SEG_V, fill, 0)

    lane = lax.iota(jnp.int32, 16)

    def scan(i, _):
        base16 = flat_v[pl.ds(i * 16, 16)]
        loc = base16 - lo
        mask = (loc >= 0) & (loc < SEG)
        loc = jnp.where(mask, loc, 0)
        ids = lane + i * 16
        plsc.store_scatter(inv_v, [loc], ids, mask=mask)
        return 0

    lax.fori_loop(0, N_V, scan, 0)

    pltpu.sync_copy(inv_v, inv_hbm.at[pl.ds(lo, SEG)])


@functools.partial(
    pl.kernel,
    out_type=jax.ShapeDtypeStruct((BS * C * NYNX,), jnp.float32),
    mesh=_MESH,
    compiler_params=_PARAMS,
    scratch_types=[
        pltpu.VMEM((SEG,), jnp.int32),
        pltpu.VMEM((NPAD,), jnp.float32),
        pltpu.VMEM((SEG,), jnp.float32),
    ],
)
def _fill_canvas(vft_hbm, inv_hbm, out_hbm, inv_v, col_v, stage_v):
    wid = _wid()
    b = wid // 8
    cell_lo = (wid % 8) * SEG
    pltpu.sync_copy(inv_hbm.at[pl.ds(wid * SEG, SEG)], inv_v)

    def chan(c, _):
        pltpu.sync_copy(vft_hbm.at[c], col_v)

        def gat(j, _):
            idx = inv_v[pl.ds(j * 16, 16)]
            stage_v[pl.ds(j * 16, 16)] = plsc.load_gather(col_v, [idx])
            return 0

        lax.fori_loop(0, SEG_V, gat, 0)
        off = (b * C + c) * NYNX + cell_lo
        pltpu.sync_copy(stage_v, out_hbm.at[pl.ds(off, SEG)])
        return 0

    lax.fori_loop(0, C, chan, 0)


def kernel(voxel_features, coors, batch_size):
    del batch_size
    flat = (coors[:, 0] * NYNX + coors[:, 2] * NX + coors[:, 3]).astype(jnp.int32)
    vft = jnp.zeros((C, NPAD), jnp.float32).at[:, :N].set(voxel_features.T)
    inv = _build_inv(flat)
    out = _fill_canvas(vft, inv)
    return out.reshape(BS, C, NY, NX)
